# baseline (speedup 1.0000x reference)
import functools

import jax
import jax.numpy as jnp
from jax import lax
from jax.experimental import pallas as pl
from jax.experimental.pallas import tpu as pltpu
from jax.experimental.pallas import tpu_sc as plsc

BATCH = 4
HGT = 14
WID = 14
HW = HGT * WID
CH = 384
RAD = 2
NOFF = 25
KK = 3
NSEL = KK * KK
NWORK = 32
RPB = 1792
ROWS_PER_W = 224
ROWS_PAD = NWORK * ROWS_PER_W
IDX_CHUNK = 56
NCHUNK = 4
CHP = 512
SL = CHP // 128
BIG = 1e30

_OFFSETS = [(di, dj) for di in range(-RAD, RAD + 1) for dj in range(-RAD, RAD + 1)]


def _select_body(xp_ref, sel_ref):
    b = pl.program_id(0)
    x = xp_ref[0]
    g = lax.dot_general(x, x, (((1,), (1,)), ((), ())),
                        preferred_element_type=jnp.float32)
    jj = lax.broadcasted_iota(jnp.int32, (HW, HW), 0)
    ll = lax.broadcasted_iota(jnp.int32, (HW, HW), 1)
    diag = jnp.sum(jnp.where(jj == ll, g, 0.0), axis=1, keepdims=True)
    gs = g * lax.rsqrt(diag)

    lcol = ll % WID
    d_rows = []
    for (di, dj) in _OFFSETS:
        off = di * WID + dj
        cond = jnp.logical_and(jj == ll + off,
                               jnp.logical_and(lcol + dj >= 0, lcol + dj < WID))
        val = jnp.sum(jnp.where(cond, gs, 0.0), axis=0, keepdims=True)
        has = jnp.sum(jnp.where(cond, 1.0, 0.0), axis=0, keepdims=True) > 0.0
        d_rows.append(jnp.where(has, val, BIG))
    d = jnp.concatenate(d_rows, axis=0)

    slot = lax.broadcasted_iota(jnp.int32, (NOFF, HW), 0)
    selmask = slot < 0
    for _ in range(NSEL):
        m = jnp.min(d, axis=0, keepdims=True)
        cand = jnp.where(d == m, slot, NOFF)
        s = jnp.min(cand, axis=0, keepdims=True)
        hit = slot == s
        selmask = jnp.logical_or(selmask, hit)
        d = jnp.where(hit, BIG, d)

    lrow = lax.broadcasted_iota(jnp.int32, (1, HW), 1)
    run = jnp.zeros((1, HW), jnp.int32)
    sel_rows = [jnp.zeros((1, HW), jnp.int32) for _ in range(NSEL)]
    for n, (di, dj) in enumerate(_OFFSETS):
        mrow = selmask[n:n + 1, :]
        gidx = lrow + (di * WID + dj)
        for p in range(NSEL):
            sel_rows[p] = jnp.where(jnp.logical_and(mrow, run == p),
                                    gidx, sel_rows[p])
        run = run + jnp.where(mrow, 1, 0)
    base = b * HW
    sel_ref[0] = jnp.concatenate(
        [sel_rows[p] + base for p in range(NSEL)], axis=0)


def _conv_body(xg_ref, wt_ref, out_ref):
    acc = jnp.dot(xg_ref[0, pl.ds(0, HW), :], wt_ref[0],
                  preferred_element_type=jnp.float32)
    for p in range(1, NSEL):
        acc = acc + jnp.dot(xg_ref[0, pl.ds(p * HW, HW), :], wt_ref[p],
                            preferred_element_type=jnp.float32)
    out_ref[0] = acc


def _sc_gather_types():
    return dict(
        out_type=jax.ShapeDtypeStruct((NWORK, ROWS_PER_W, CHP // 2),
                                      jnp.int32),
        scratch_types=[
            pltpu.VMEM((NCHUNK, IDX_CHUNK), jnp.int32),
            pltpu.VMEM((ROWS_PER_W, CHP // 2), jnp.int32),
            pltpu.SemaphoreType.DMA,
            pltpu.SemaphoreType.DMA,
        ],
    )


def _sc_gather(table, selw):
    mesh = plsc.VectorSubcoreMesh(core_axis_name="c", subcore_axis_name="s")

    @functools.partial(pl.kernel, mesh=mesh, **_sc_gather_types())
    def k(table_hbm, sel_hbm, out_hbm, idx_v, rows_v, sem, sem2):
        wid = lax.axis_index("s") * 2 + lax.axis_index("c")
        pltpu.sync_copy(sel_hbm.at[wid], idx_v)
        gathers = [
            pltpu.async_copy(table_hbm.at[idx_v.at[i]],
                             rows_v.at[pl.ds(i * IDX_CHUNK, IDX_CHUNK)], sem)
            for i in range(NCHUNK)
        ]
        stores = []
        for i in range(NCHUNK):
            gathers[i].wait()
            stores.append(
                pltpu.async_copy(
                    rows_v.at[pl.ds(i * IDX_CHUNK, IDX_CHUNK)],
                    out_hbm.at[wid].at[pl.ds(i * IDX_CHUNK, IDX_CHUNK)],
                    sem2))
        for cp in stores:
            cp.wait()

    return k(table, selw)


def kernel(x, Wc):
    xp = x.reshape(BATCH, CH, HW).transpose(0, 2, 1)
    wt = jnp.transpose(Wc.reshape(CH, CH, NSEL), (2, 1, 0))
    wtp = jnp.pad(wt, ((0, 0), (0, CHP - CH), (0, 0))).astype(jnp.bfloat16)
    xbf = jnp.pad(xp.reshape(BATCH * HW, CH),
                  ((0, 0), (0, CHP - CH))).astype(jnp.bfloat16)
    sel = pl.pallas_call(
        _select_body,
        grid=(BATCH,),
        in_specs=[pl.BlockSpec((1, HW, CH), lambda i: (i, 0, 0))],
        out_specs=pl.BlockSpec((1, NSEL, HW), lambda i: (i, 0, 0)),
        out_shape=jax.ShapeDtypeStruct((BATCH, NSEL, HW), jnp.int32),
    )(xp)
    selw = jnp.pad(sel.reshape(BATCH, NSEL * HW), ((0, 0), (0, RPB - NSEL * HW)))
    xpack = lax.bitcast_convert_type(
        xbf.reshape(BATCH * HW, CHP // 2, 2), jnp.int32)
    xgp = _sc_gather(xpack, selw.reshape(NWORK, NCHUNK, IDX_CHUNK))
    xg = lax.bitcast_convert_type(
        xgp.reshape(ROWS_PAD, CHP // 2), jnp.bfloat16).reshape(ROWS_PAD, CHP)
    out = pl.pallas_call(
        _conv_body,
        grid=(BATCH,),
        in_specs=[
            pl.BlockSpec((1, RPB, CHP), lambda i: (i, 0, 0)),
            pl.BlockSpec((NSEL, CHP, CH), lambda i: (0, 0, 0)),
        ],
        out_specs=pl.BlockSpec((1, HW, CH), lambda i: (i, 0, 0)),
        out_shape=jax.ShapeDtypeStruct((BATCH, HW, CH), jnp.float32),
    )(xg.reshape(BATCH, RPB, CHP), wtp)
    out = out.transpose(0, 2, 1)
    return out.reshape(BATCH, CH, HGT, WID)


# speedup vs baseline: 2.1793x; 2.1793x over previous
import functools

import jax
import jax.numpy as jnp
from jax import lax
from jax.experimental import pallas as pl
from jax.experimental.pallas import tpu as pltpu
from jax.experimental.pallas import tpu_sc as plsc

BATCH = 4
HGT = 14
WID = 14
HW = HGT * WID
CH = 384
RAD = 2
NOFF = 25
KK = 3
NSEL = KK * KK
NWORK = 32
RPB = 1792
ROWS_PER_W = 224
ROWS_PAD = NWORK * ROWS_PER_W
IDX_CHUNK = 56
NCHUNK = 4
BIG = 1e30

_OFFSETS = [(di, dj) for di in range(-RAD, RAD + 1) for dj in range(-RAD, RAD + 1)]


def _select_body(xin_ref, sel_ref, xp_ref):
    b = pl.program_id(0)
    xin = xin_ref[0]
    g = lax.dot_general(xin, xin, (((0,), (0,)), ((), ())),
                        preferred_element_type=jnp.float32)
    xp_ref[0] = xin.T
    jj = lax.broadcasted_iota(jnp.int32, (HW, HW), 0)
    ll = lax.broadcasted_iota(jnp.int32, (HW, HW), 1)
    diag = jnp.sum(jnp.where(jj == ll, g, 0.0), axis=1, keepdims=True)
    gs = g * lax.rsqrt(diag)

    lcol = ll % WID
    d_rows = []
    for (di, dj) in _OFFSETS:
        off = di * WID + dj
        cond = jnp.logical_and(jj == ll + off,
                               jnp.logical_and(lcol + dj >= 0, lcol + dj < WID))
        val = jnp.sum(jnp.where(cond, gs, 0.0), axis=0, keepdims=True)
        has = jnp.sum(jnp.where(cond, 1.0, 0.0), axis=0, keepdims=True) > 0.0
        d_rows.append(jnp.where(has, val, BIG))
    d = jnp.concatenate(d_rows, axis=0)

    slot = lax.broadcasted_iota(jnp.int32, (NOFF, HW), 0)
    selmask = slot < 0
    for _ in range(NSEL):
        m = jnp.min(d, axis=0, keepdims=True)
        cand = jnp.where(d == m, slot, NOFF)
        s = jnp.min(cand, axis=0, keepdims=True)
        hit = slot == s
        selmask = jnp.logical_or(selmask, hit)
        d = jnp.where(hit, BIG, d)

    lrow = lax.broadcasted_iota(jnp.int32, (1, HW), 1)
    run = jnp.zeros((1, HW), jnp.int32)
    sel_rows = [jnp.zeros((1, HW), jnp.int32) for _ in range(NSEL)]
    for n, (di, dj) in enumerate(_OFFSETS):
        mrow = selmask[n:n + 1, :]
        gidx = lrow + (di * WID + dj)
        for p in range(NSEL):
            sel_rows[p] = jnp.where(jnp.logical_and(mrow, run == p),
                                    gidx, sel_rows[p])
        run = run + jnp.where(mrow, 1, 0)
    base = b * HW
    sel_ref[0] = jnp.concatenate(
        [sel_rows[p] + base for p in range(NSEL)], axis=0)


def _conv_body(xg_ref, wt_ref, out_ref):
    acc = None
    for p in range(NSEL):
        xgp = xg_ref[0, pl.ds(p * HW, HW), :].astype(jnp.bfloat16)
        part = lax.dot_general(wt_ref[p], xgp, (((1,), (1,)), ((), ())),
                               preferred_element_type=jnp.float32)
        acc = part if acc is None else acc + part
    out_ref[0] = acc


def _sc_gather_types():
    return dict(
        out_type=jax.ShapeDtypeStruct((NWORK, ROWS_PER_W, CH), jnp.float32),
        scratch_types=[
            pltpu.VMEM((NCHUNK, IDX_CHUNK), jnp.int32),
            pltpu.VMEM((ROWS_PER_W, CH), jnp.float32),
            pltpu.SemaphoreType.DMA,
            pltpu.SemaphoreType.DMA,
        ],
    )


def _sc_gather(table, selw):
    mesh = plsc.VectorSubcoreMesh(core_axis_name="c", subcore_axis_name="s")

    @functools.partial(pl.kernel, mesh=mesh, **_sc_gather_types())
    def k(table_hbm, sel_hbm, out_hbm, idx_v, rows_v, sem, sem2):
        wid = lax.axis_index("s") * 2 + lax.axis_index("c")
        pltpu.sync_copy(sel_hbm.at[wid], idx_v)
        gathers = [
            pltpu.async_copy(table_hbm.at[idx_v.at[i]],
                             rows_v.at[pl.ds(i * IDX_CHUNK, IDX_CHUNK)], sem)
            for i in range(NCHUNK)
        ]
        stores = []
        for i in range(NCHUNK):
            gathers[i].wait()
            stores.append(
                pltpu.async_copy(
                    rows_v.at[pl.ds(i * IDX_CHUNK, IDX_CHUNK)],
                    out_hbm.at[wid].at[pl.ds(i * IDX_CHUNK, IDX_CHUNK)],
                    sem2))
        for cp in stores:
            cp.wait()

    return k(table, selw)


def kernel(x, Wc):
    xin = x.reshape(BATCH, CH, HW)
    wt = jnp.transpose(Wc.reshape(CH, CH, NSEL),
                       (2, 0, 1)).astype(jnp.bfloat16)
    sel, xp = pl.pallas_call(
        _select_body,
        grid=(BATCH,),
        in_specs=[pl.BlockSpec((1, CH, HW), lambda i: (i, 0, 0))],
        out_specs=[
            pl.BlockSpec((1, NSEL, HW), lambda i: (i, 0, 0)),
            pl.BlockSpec((1, HW, CH), lambda i: (i, 0, 0)),
        ],
        out_shape=[
            jax.ShapeDtypeStruct((BATCH, NSEL, HW), jnp.int32),
            jax.ShapeDtypeStruct((BATCH, HW, CH), jnp.float32),
        ],
    )(xin)
    selw = jnp.pad(sel.reshape(BATCH, NSEL * HW), ((0, 0), (0, RPB - NSEL * HW)))
    xg = _sc_gather(xp.reshape(BATCH * HW, CH),
                    selw.reshape(NWORK, NCHUNK, IDX_CHUNK))
    out = pl.pallas_call(
        _conv_body,
        grid=(BATCH,),
        in_specs=[
            pl.BlockSpec((1, RPB, CH), lambda i: (i, 0, 0)),
            pl.BlockSpec((NSEL, CH, CH), lambda i: (0, 0, 0)),
        ],
        out_specs=pl.BlockSpec((1, CH, HW), lambda i: (i, 0, 0)),
        out_shape=jax.ShapeDtypeStruct((BATCH, CH, HW), jnp.float32),
    )(xg.reshape(BATCH, RPB, CH), wt)
    return out.reshape(BATCH, CH, HGT, WID)


# speedup vs baseline: 2.3391x; 1.0733x over previous
import functools

import jax
import jax.numpy as jnp
from jax import lax
from jax.experimental import pallas as pl
from jax.experimental.pallas import tpu as pltpu
from jax.experimental.pallas import tpu_sc as plsc

BATCH = 4
HGT = 14
WID = 14
HW = HGT * WID
CH = 384
RAD = 2
NOFF = 25
KK = 3
NSEL = KK * KK
NWORK = 32
RPB = 1792
ROWS_PER_W = 224
ROWS_PAD = NWORK * ROWS_PER_W
IDX_CHUNK = 56
NCHUNK = 4
PCHW = CH // 2
CHW = 256
BIG = 1e30

_OFFSETS = [(di, dj) for di in range(-RAD, RAD + 1) for dj in range(-RAD, RAD + 1)]


def _select_body(xin_ref, sel_ref, xp_ref):
    b = pl.program_id(0)
    xin = xin_ref[0]
    g = lax.dot_general(xin, xin, (((0,), (0,)), ((), ())),
                        preferred_element_type=jnp.float32)
    packed = pltpu.bitcast(xin.astype(jnp.bfloat16), jnp.int32).T
    xp_ref[0] = jnp.concatenate(
        [packed, jnp.zeros((HW, CHW - PCHW), jnp.int32)], axis=1)
    jj = lax.broadcasted_iota(jnp.int32, (HW, HW), 0)
    ll = lax.broadcasted_iota(jnp.int32, (HW, HW), 1)
    diag = jnp.sum(jnp.where(jj == ll, g, 0.0), axis=1, keepdims=True)
    gs = g * lax.rsqrt(diag)

    lcol = ll % WID
    d_rows = []
    for (di, dj) in _OFFSETS:
        off = di * WID + dj
        cond = jnp.logical_and(jj == ll + off,
                               jnp.logical_and(lcol + dj >= 0, lcol + dj < WID))
        val = jnp.sum(jnp.where(cond, gs, 0.0), axis=0, keepdims=True)
        has = jnp.sum(jnp.where(cond, 1.0, 0.0), axis=0, keepdims=True) > 0.0
        d_rows.append(jnp.where(has, val, BIG))
    d = jnp.concatenate(d_rows, axis=0)

    slot = lax.broadcasted_iota(jnp.int32, (NOFF, HW), 0)
    selmask = slot < 0
    for _ in range(NSEL):
        m = jnp.min(d, axis=0, keepdims=True)
        cand = jnp.where(d == m, slot, NOFF)
        s = jnp.min(cand, axis=0, keepdims=True)
        hit = slot == s
        selmask = jnp.logical_or(selmask, hit)
        d = jnp.where(hit, BIG, d)

    lrow = lax.broadcasted_iota(jnp.int32, (1, HW), 1)
    run = jnp.zeros((1, HW), jnp.int32)
    sel_rows = [jnp.zeros((1, HW), jnp.int32) for _ in range(NSEL)]
    for n, (di, dj) in enumerate(_OFFSETS):
        mrow = selmask[n:n + 1, :]
        gidx = lrow + (di * WID + dj)
        for p in range(NSEL):
            sel_rows[p] = jnp.where(jnp.logical_and(mrow, run == p),
                                    gidx, sel_rows[p])
        run = run + jnp.where(mrow, 1, 0)
    base = b * HW
    sel_ref[0] = jnp.concatenate(
        [sel_rows[p] + base for p in range(NSEL)], axis=0)


def _conv_body(xg_ref, wt_ref, out_ref):
    acc = None
    for p in range(NSEL):
        xgp = pltpu.bitcast(xg_ref[0, pl.ds(p * HW, HW), :].T,
                            jnp.bfloat16)[0:CH]
        part = lax.dot_general(wt_ref[p], xgp, (((1,), (0,)), ((), ())),
                               preferred_element_type=jnp.float32)
        acc = part if acc is None else acc + part
    out_ref[0] = acc


def _sc_gather_types():
    return dict(
        out_type=jax.ShapeDtypeStruct((NWORK, ROWS_PER_W, CHW), jnp.int32),
        scratch_types=[
            pltpu.VMEM((NCHUNK, IDX_CHUNK), jnp.int32),
            pltpu.VMEM((ROWS_PER_W, CHW), jnp.int32),
            pltpu.SemaphoreType.DMA,
            pltpu.SemaphoreType.DMA,
        ],
    )


def _sc_gather(table, selw):
    mesh = plsc.VectorSubcoreMesh(core_axis_name="c", subcore_axis_name="s")

    @functools.partial(pl.kernel, mesh=mesh, **_sc_gather_types())
    def k(table_hbm, sel_hbm, out_hbm, idx_v, rows_v, sem, sem2):
        wid = lax.axis_index("s") * 2 + lax.axis_index("c")
        pltpu.sync_copy(sel_hbm.at[wid], idx_v)
        gathers = [
            pltpu.async_copy(table_hbm.at[idx_v.at[i]],
                             rows_v.at[pl.ds(i * IDX_CHUNK, IDX_CHUNK)], sem)
            for i in range(NCHUNK)
        ]
        stores = []
        for i in range(NCHUNK):
            gathers[i].wait()
            stores.append(
                pltpu.async_copy(
                    rows_v.at[pl.ds(i * IDX_CHUNK, IDX_CHUNK)],
                    out_hbm.at[wid].at[pl.ds(i * IDX_CHUNK, IDX_CHUNK)],
                    sem2))
        for cp in stores:
            cp.wait()

    return k(table, selw)


def kernel(x, Wc):
    xin = x.reshape(BATCH, CH, HW)
    wt = jnp.transpose(Wc.reshape(CH, CH, NSEL),
                       (2, 0, 1)).astype(jnp.bfloat16)
    sel, xp = pl.pallas_call(
        _select_body,
        grid=(BATCH,),
        in_specs=[pl.BlockSpec((1, CH, HW), lambda i: (i, 0, 0))],
        out_specs=[
            pl.BlockSpec((1, NSEL, HW), lambda i: (i, 0, 0)),
            pl.BlockSpec((1, HW, CHW), lambda i: (i, 0, 0)),
        ],
        out_shape=[
            jax.ShapeDtypeStruct((BATCH, NSEL, HW), jnp.int32),
            jax.ShapeDtypeStruct((BATCH, HW, CHW), jnp.int32),
        ],
    )(xin)
    selw = jnp.pad(sel.reshape(BATCH, NSEL * HW), ((0, 0), (0, RPB - NSEL * HW)))
    xg = _sc_gather(xp.reshape(BATCH * HW, CHW),
                    selw.reshape(NWORK, NCHUNK, IDX_CHUNK))
    out = pl.pallas_call(
        _conv_body,
        grid=(BATCH,),
        in_specs=[
            pl.BlockSpec((1, RPB, CHW), lambda i: (i, 0, 0)),
            pl.BlockSpec((NSEL, CH, CH), lambda i: (0, 0, 0)),
        ],
        out_specs=pl.BlockSpec((1, CH, HW), lambda i: (i, 0, 0)),
        out_shape=jax.ShapeDtypeStruct((BATCH, CH, HW), jnp.float32),
    )(xg.reshape(BATCH, RPB, CHW), wt)
    return out.reshape(BATCH, CH, HGT, WID)
